# Initial kernel scaffold; baseline (speedup 1.0000x reference)
import jax
import jax.numpy as jnp
from jax.experimental import pallas as pl


def kernel(x, edge_index, W1, b1, W2, b2):
    raise NotImplementedError("write your pallas kernel here")


# speedup vs baseline: 9.1505x; 9.1505x over previous
import functools

import jax
import jax.numpy as jnp
from jax import lax
from jax.experimental import pallas as pl
from jax.experimental.pallas import tpu as pltpu
from jax.experimental.pallas import tpu_sc as plsc

N_NODES = 10000
D = 128
E = 320000

NC = 2
NS = 16
NW = NC * NS

NP = 10240
PAD_NODE = 10200

CHUNK = 64
EPW_CHUNKS = 160
EPW = CHUNK * EPW_CHUNKS
EP = EPW * NW
ROWS_PER_TILE = NP // NS

_MESH = plsc.VectorSubcoreMesh(core_axis_name="c", subcore_axis_name="s")
_SC_PARAMS = pltpu.CompilerParams(needs_layout_passes=False)


@functools.partial(
    pl.kernel,
    out_type=jax.ShapeDtypeStruct((NW, NP), jnp.float32),
    mesh=_MESH,
    compiler_params=_SC_PARAMS,
    scratch_types=[
        pltpu.VMEM((EPW,), jnp.int32),
        pltpu.VMEM((NP,), jnp.float32),
    ],
)
def _hist_sc(dst_hbm, out_hbm, dst_v, hist_v):
    wid = lax.axis_index("s") * NC + lax.axis_index("c")
    pltpu.sync_copy(dst_hbm.at[wid], dst_v)

    def _zero(i, carry):
        hist_v[pl.ds(i * 16, 16)] = jnp.zeros((16,), jnp.float32)
        return carry

    lax.fori_loop(0, NP // 16, _zero, 0)

    ones = jnp.ones((16,), jnp.float32)

    def _accum(i, carry):
        idx = dst_v[pl.ds(i * 16, 16)]
        plsc.addupdate_scatter(hist_v, [idx], ones)
        return carry

    lax.fori_loop(0, EPW // 16, _accum, 0)
    pltpu.sync_copy(hist_v, out_hbm.at[wid])


@functools.partial(
    pl.kernel,
    out_type=jax.ShapeDtypeStruct((NC, NP, D), jnp.float32),
    mesh=_MESH,
    compiler_params=_SC_PARAMS,
    scratch_types=[
        pltpu.VMEM((EPW_CHUNKS // 2, CHUNK), jnp.int32),
        pltpu.VMEM((EPW_CHUNKS // 2, CHUNK), jnp.int32),
        pltpu.VMEM((2, CHUNK, D), jnp.float32),
        pltpu.VMEM_SHARED((NP, D), jnp.float32),
        pltpu.SemaphoreType.DMA,
        pltpu.SemaphoreType.DMA,
    ],
)
def _edge_sc(hp_hbm, src_hbm, dst_hbm, out_hbm,
             src_v, dst_v, rows_v, acc, sem0, sem1):
    cid = lax.axis_index("c")
    sid = lax.axis_index("s")
    wid = sid * NC + cid

    def _zrow(i, carry):
        for c in range(D // 16):
            rows_v[0, i, pl.ds(c * 16, 16)] = jnp.zeros((16,), jnp.float32)
        return carry

    lax.fori_loop(0, CHUNK, _zrow, 0)
    base = sid * ROWS_PER_TILE
    for r in range(ROWS_PER_TILE // CHUNK):
        pltpu.sync_copy(rows_v.at[0], acc.at[pl.ds(base + r * CHUNK, CHUNK)])
    plsc.subcore_barrier()

    sems = (sem0, sem1)
    half = EPW_CHUNKS // 2
    for h in range(2):
        pltpu.sync_copy(src_hbm.at[wid, pl.ds(h * half, half)], src_v)
        pltpu.sync_copy(dst_hbm.at[wid, pl.ds(h * half, half)], dst_v)
        pltpu.async_copy(hp_hbm.at[src_v.at[0]], rows_v.at[0], sem0)

        def _chunks(jj, carry):
            for b in range(2):
                j = jj * 2 + b
                nxt = j + 1

                @pl.when(nxt < half)
                def _():
                    pltpu.async_copy(hp_hbm.at[src_v.at[nxt]],
                                     rows_v.at[1 - b], sems[1 - b])

                pltpu.make_async_copy(hp_hbm.at[src_v.at[j]],
                                      rows_v.at[b], sems[b]).wait()
                pltpu.sync_copy(rows_v.at[b], acc.at[dst_v.at[j]], add=True)
            return carry

        lax.fori_loop(0, half // 2, _chunks, 0)
    plsc.subcore_barrier()
    pltpu.sync_copy(acc.at[pl.ds(base, ROWS_PER_TILE)],
                    out_hbm.at[cid, pl.ds(base, ROWS_PER_TILE)])


_R = 256


def _deg_scale(hist_blk, i):
    deg = 1.0 + jnp.sum(hist_blk, axis=0)
    dis = (1.0 / jnp.sqrt(deg))[:, None]
    rows = i * _R + lax.broadcasted_iota(jnp.int32, (_R, 1), 0)
    return jnp.where(rows < N_NODES, dis, 0.0)


def _prep_body(x_ref, w_ref, hist_ref, o_ref):
    i = pl.program_id(0)
    h = jnp.dot(x_ref[...], w_ref[...], preferred_element_type=jnp.float32)
    o_ref[...] = h * _deg_scale(hist_ref[...], i)


def _mid_body(p_ref, hp_ref, hist_ref, b_ref, w_ref, o_ref):
    i = pl.program_id(0)
    d2 = _deg_scale(hist_ref[...], i)
    s = p_ref[0] + p_ref[1] + hp_ref[...]
    z = jnp.maximum(s * d2 + b_ref[...], 0.0)
    o_ref[...] = jnp.dot(z, w_ref[...], preferred_element_type=jnp.float32) * d2


def _final_body(p_ref, hp_ref, hist_ref, b_ref, o_ref):
    i = pl.program_id(0)
    d2 = _deg_scale(hist_ref[...], i)
    s = p_ref[0] + p_ref[1] + hp_ref[...]
    o_ref[...] = s * d2 + b_ref[...]


_ROWS_SPEC = pl.BlockSpec((_R, D), lambda i: (i, 0))
_HIST_SPEC = pl.BlockSpec((NW, _R), lambda i: (0, i))
_FULL_W = pl.BlockSpec((D, D), lambda i: (0, 0))
_BIAS_SPEC = pl.BlockSpec((1, D), lambda i: (0, 0))
_PARTS_SPEC = pl.BlockSpec((NC, _R, D), lambda i: (0, i, 0))
_OUT_SDS = jax.ShapeDtypeStruct((NP, D), jnp.float32)

_prep_tc = pl.pallas_call(
    _prep_body, grid=(NP // _R,),
    in_specs=[_ROWS_SPEC, _FULL_W, _HIST_SPEC],
    out_specs=_ROWS_SPEC, out_shape=_OUT_SDS)

_mid_tc = pl.pallas_call(
    _mid_body, grid=(NP // _R,),
    in_specs=[_PARTS_SPEC, _ROWS_SPEC, _HIST_SPEC, _BIAS_SPEC, _FULL_W],
    out_specs=_ROWS_SPEC, out_shape=_OUT_SDS)

_final_tc = pl.pallas_call(
    _final_body, grid=(NP // _R,),
    in_specs=[_PARTS_SPEC, _ROWS_SPEC, _HIST_SPEC, _BIAS_SPEC],
    out_specs=_ROWS_SPEC, out_shape=_OUT_SDS)


def kernel(x, edge_index, W1, b1, W2, b2):
    src = edge_index[0].astype(jnp.int32)
    dst = edge_index[1].astype(jnp.int32)
    pad = jnp.full((EP - E,), PAD_NODE, jnp.int32)
    src_p = jnp.concatenate([src, pad]).reshape(NW, EPW_CHUNKS, CHUNK)
    dst_p = jnp.concatenate([dst, pad]).reshape(NW, EPW_CHUNKS, CHUNK)
    xp = jnp.zeros((NP, D), jnp.float32).at[:N_NODES].set(x)
    b1r = b1.reshape(1, D)
    b2r = b2.reshape(1, D)

    hist = _hist_sc(dst_p.reshape(NW, EPW))
    hp1 = _prep_tc(xp, W1, hist)
    parts1 = _edge_sc(hp1, src_p, dst_p)
    hp2 = _mid_tc(parts1, hp1, hist, b1r, W2)
    parts2 = _edge_sc(hp2, src_p, dst_p)
    zp = _final_tc(parts2, hp2, hist, b2r)
    return zp[:N_NODES]
